# Initial kernel scaffold; baseline (speedup 1.0000x reference)
import jax
import jax.numpy as jnp
from jax.experimental import pallas as pl


def kernel(ufeats, ifeats, edge_index, edge_type, W_ui0, W_iu0, Wu0, bu0, Wi0, bi0, W_ui1, W_iu1, Wu1, bu1, Wi1, bi1, W_h, b_h):
    raise NotImplementedError("write your pallas kernel here")


# speedup vs baseline: 8.7174x; 8.7174x over previous
import functools

import jax
import jax.numpy as jnp
from jax import lax
from jax.experimental import pallas as pl
from jax.experimental.pallas import tpu as pltpu
from jax.experimental.pallas import tpu_sc as plsc

N_U = 10000
N_I = 10000
R = 5
D = 128
E = 320000
RN = R * N_U

NC = 2
NS = 16

CH = 128
CPT = 160
WIN = 8
NWIN = CPT // WIN
EPT = CH * CPT
EPAD = EPT * NS

ZCH = 400
NZ = 8
ACC_PT = NZ * ZCH
ACC_ROWS = ACC_PT * NS
TRASH = 50200
OUT_PT = 3128
OUT_LAST = RN - 15 * OUT_PT

NB = 25
BN = N_U // NB

_mesh = plsc.VectorSubcoreMesh(core_axis_name="c", subcore_axis_name="s")
_sc_params = pltpu.CompilerParams(use_tc_tiling_on_sc=False)


def _deg_body(sA, sB, degu_out, degi_out, idx_v, ones_v, zero_v, acc):
    cid = lax.axis_index("c")
    tid = lax.axis_index("s")

    one16 = jnp.ones((16,), jnp.float32)
    z16 = jnp.zeros((16,), jnp.float32)

    @pl.loop(0, CH)
    def _(i):
        ones_v[i, :] = one16

    @pl.loop(0, ZCH)
    def _(i):
        zero_v[i, :] = z16

    for k in range(NZ):
        pltpu.sync_copy(zero_v, acc.at[pl.ds(tid * ACC_PT + k * ZCH, ZCH)])

    @pl.when(cid == 0)
    def _():
        pltpu.sync_copy(sA.at[tid], idx_v)

    @pl.when(cid == 1)
    def _():
        pltpu.sync_copy(sB.at[tid], idx_v)

    plsc.subcore_barrier()

    @pl.loop(0, CPT)
    def _(ci):
        pltpu.sync_copy(ones_v, acc.at[idx_v.at[ci]], add=True)

    plsc.subcore_barrier()

    for c, out in ((0, degu_out), (1, degi_out)):

        @pl.when(jnp.logical_and(cid == c, tid < 15))
        def _(out=out):
            pltpu.sync_copy(acc.at[pl.ds(tid * OUT_PT, OUT_PT)],
                            out.at[pl.ds(tid * OUT_PT, OUT_PT)])

        @pl.when(jnp.logical_and(cid == c, tid == 15))
        def _(out=out):
            pltpu.sync_copy(acc.at[pl.ds(15 * OUT_PT, OUT_LAST)],
                            out.at[pl.ds(15 * OUT_PT, OUT_LAST)])


_deg_call = pl.kernel(
    _deg_body,
    out_type=[jax.ShapeDtypeStruct((RN, 16), jnp.float32),
              jax.ShapeDtypeStruct((RN, 16), jnp.float32)],
    mesh=_mesh,
    scratch_types=[
        pltpu.VMEM((CPT, CH), jnp.int32),
        pltpu.VMEM((CH, 16), jnp.float32),
        pltpu.VMEM((ZCH, 16), jnp.float32),
        pltpu.VMEM_SHARED((ACC_ROWS, 16), jnp.float32),
    ],
    compiler_params=_sc_params,
)


def _agg_body(u0, u1, u2, u3, i0, i1, i2, i3, gA, sA, gB, sB,
              ai0, ai1, ai2, ai3, au0, au1, au2, au3,
              gwin_v, swin_v, rows_v, zero_v, acc, sem):
    cid = lax.axis_index("c")
    tid = lax.axis_index("s")

    z16 = jnp.zeros((16,), jnp.float32)

    @pl.loop(0, ZCH)
    def _(i):
        zero_v[i, pl.ds(0, 16)] = z16
        zero_v[i, pl.ds(16, 16)] = z16

    for d in range(2):
        tabs = (u0, u1, u2, u3) if d == 0 else (i0, i1, i2, i3)
        outs = (ai0, ai1, ai2, ai3) if d == 0 else (au0, au1, au2, au3)
        g_h = gA if d == 0 else gB
        s_h = sB if d == 0 else sA
        for p in range(4):

            @pl.when(cid == p // 2)
            def _(tab=tabs[p], out=outs[p], g_h=g_h, s_h=s_h):
                for k in range(NZ):
                    pltpu.sync_copy(
                        zero_v, acc.at[pl.ds(tid * ACC_PT + k * ZCH, ZCH)])
                plsc.subcore_barrier()

                @pl.loop(0, NWIN)
                def _(w):
                    pltpu.sync_copy(
                        g_h.at[tid].at[pl.ds(w * WIN, WIN)], gwin_v)
                    pltpu.sync_copy(
                        s_h.at[tid].at[pl.ds(w * WIN, WIN)], swin_v)

                    @pl.loop(0, WIN)
                    def _(j):
                        pltpu.async_copy(
                            tab.at[gwin_v.at[j]], rows_v, sem).wait()
                        pltpu.sync_copy(
                            rows_v, acc.at[swin_v.at[j]], add=True)

                plsc.subcore_barrier()

                @pl.when(tid < 15)
                def _():
                    pltpu.sync_copy(acc.at[pl.ds(tid * OUT_PT, OUT_PT)],
                                    out.at[pl.ds(tid * OUT_PT, OUT_PT)])

                @pl.when(tid == 15)
                def _():
                    pltpu.sync_copy(acc.at[pl.ds(15 * OUT_PT, OUT_LAST)],
                                    out.at[pl.ds(15 * OUT_PT, OUT_LAST)])

                plsc.subcore_barrier()


_agg_call = pl.kernel(
    _agg_body,
    out_type=[jax.ShapeDtypeStruct((RN, 32), jnp.float32)] * 8,
    mesh=_mesh,
    scratch_types=[
        pltpu.VMEM((WIN, CH), jnp.int32),
        pltpu.VMEM((WIN, CH), jnp.int32),
        pltpu.VMEM((CH, 32), jnp.float32),
        pltpu.VMEM((ZCH, 32), jnp.float32),
        pltpu.VMEM_SHARED((ACC_ROWS, 32), jnp.float32),
        pltpu.SemaphoreType.DMA,
    ],
    compiler_params=_sc_params,
)


def _scale_body(f_ref, w_ref, deg_ref, o0, o1, o2, o3):
    cs = lax.rsqrt(jnp.maximum(deg_ref[0, :, 0:1], 1.0))
    h = jnp.dot(f_ref[...], w_ref[0],
                preferred_element_type=jnp.float32) * cs
    o0[0] = h[:, 0:32]
    o1[0] = h[:, 32:64]
    o2[0] = h[:, 64:96]
    o3[0] = h[:, 96:128]


def _scale(feats, W, deg3):
    return pl.pallas_call(
        _scale_body,
        grid=(R, NB),
        in_specs=[
            pl.BlockSpec((BN, D), lambda r, n: (n, 0)),
            pl.BlockSpec((1, D, D), lambda r, n: (r, 0, 0)),
            pl.BlockSpec((1, BN, 16), lambda r, n: (r, n, 0)),
        ],
        out_specs=[pl.BlockSpec((1, BN, 32), lambda r, n: (r, n, 0))] * 4,
        out_shape=[jax.ShapeDtypeStruct((R, N_U, 32), jnp.float32)] * 4,
    )(feats, W, deg3)


def _combine_body(s0, s1, s2, s3, deg_ref, w_ref, b_ref, o_ref):
    cs = lax.rsqrt(jnp.maximum(deg_ref[:, :, 0:1], 1.0))
    parts = [jnp.sum(s[...] * cs, axis=0) for s in (s0, s1, s2, s3)]
    x = jnp.maximum(jnp.concatenate(parts, axis=1), 0.0)
    h = jnp.dot(x, w_ref[...], preferred_element_type=jnp.float32) + b_ref[0]
    o_ref[...] = jnp.maximum(h, 0.0)


def _combine(slices, deg3, W, b):
    return pl.pallas_call(
        _combine_body,
        grid=(NB,),
        in_specs=[pl.BlockSpec((R, BN, 32), lambda n: (0, n, 0))] * 4 + [
            pl.BlockSpec((R, BN, 16), lambda n: (0, n, 0)),
            pl.BlockSpec((D, D), lambda n: (0, 0)),
            pl.BlockSpec((1, D), lambda n: (0, 0)),
        ],
        out_specs=pl.BlockSpec((BN, D), lambda n: (n, 0)),
        out_shape=jax.ShapeDtypeStruct((N_U, D), jnp.float32),
    )(*slices, deg3, W, b.reshape(1, D))


def _dense_body(x_ref, w_ref, b_ref, o_ref):
    o_ref[...] = jnp.dot(x_ref[...], w_ref[...],
                         preferred_element_type=jnp.float32) + b_ref[0]


def _dense(x, W, b):
    return pl.pallas_call(
        _dense_body,
        grid=(NB,),
        in_specs=[
            pl.BlockSpec((BN, D), lambda n: (n, 0)),
            pl.BlockSpec((D, D), lambda n: (0, 0)),
            pl.BlockSpec((1, D), lambda n: (0, 0)),
        ],
        out_specs=pl.BlockSpec((BN, D), lambda n: (n, 0)),
        out_shape=jax.ShapeDtypeStruct((N_U, D), jnp.float32),
    )(x, W, b.reshape(1, D))


def kernel(ufeats, ifeats, edge_index, edge_type,
           W_ui0, W_iu0, Wu0, bu0, Wi0, bi0,
           W_ui1, W_iu1, Wu1, bu1, Wi1, bi1, W_h, b_h):
    src = edge_index[0]
    dst = edge_index[1]
    et = edge_type

    A = et * N_U + src
    B = et * N_I + dst
    pad_g = jnp.zeros((EPAD - E,), jnp.int32)
    pad_s = jnp.full((EPAD - E,), TRASH, jnp.int32)
    gA = jnp.concatenate([A, pad_g]).reshape(NS, CPT, CH)
    sA = jnp.concatenate([A, pad_s]).reshape(NS, CPT, CH)
    gB = jnp.concatenate([B, pad_g]).reshape(NS, CPT, CH)
    sB = jnp.concatenate([B, pad_s]).reshape(NS, CPT, CH)

    deg_u, deg_i = _deg_call(sA, sB)
    degu3 = deg_u.reshape(R, N_U, 16)
    degi3 = deg_i.reshape(R, N_I, 16)

    uf, itf = ufeats, ifeats
    for (W_ui, W_iu, Wu, bu, Wi, bi) in (
            (W_ui0, W_iu0, Wu0, bu0, Wi0, bi0),
            (W_ui1, W_iu1, Wu1, bu1, Wi1, bi1)):
        u_sl = _scale(uf, W_ui, degu3)
        i_sl = _scale(itf, W_iu, degi3)
        flat = [t.reshape(RN, 32) for t in (*u_sl, *i_sl)]
        aggs = _agg_call(*flat, gA, sA, gB, sB)
        ai = [t.reshape(R, N_I, 32) for t in aggs[:4]]
        au = [t.reshape(R, N_U, 32) for t in aggs[4:]]
        itf = _combine(ai, degi3, Wi, bi)
        uf = _combine(au, degu3, Wu, bu)

    out_u = _dense(uf, W_h, b_h)
    out_i = _dense(itf, W_h, b_h)
    return (out_u, out_i)
